# baseline (speedup 1.0000x reference)
import jax
import jax.numpy as jnp
from jax import lax
from jax.experimental import pallas as pl
from jax.experimental.pallas import tpu as pltpu
from jax.experimental.pallas import tpu_sc as plsc

N = 10000
E = 160000
D = 256
NC = 2
NS = 16
LANES = 16
HW = 128
CW = 64
EPW = E // NS
EB = 40
STG = 2000
NPPS = STG // (2 * EB)
NPAIR = EPW // (2 * EB)
NPW = N // NS
_TAIL = (0, 16, 24)

_NODE_BLK = 1000
_EDGE_BLK = 2000


def _node_proj_body(x_ref, wg_ref, bg_ref, wsu_ref, bsu_ref, t_ref, xsu_ref):
    x = x_ref[...]
    g = jnp.dot(x, wg_ref[...], preferred_element_type=jnp.float32) + bg_ref[...]
    for q in range(3):
        for h in range(NC):
            t_ref[q * NC + h] = g[:, q * D + h * HW:q * D + (h + 1) * HW]
    xsu_ref[...] = (
        jnp.dot(x, wsu_ref[...], preferred_element_type=jnp.float32) + bsu_ref[...]
    )


def _edge_proj_body(x_ref, w_ref, b_ref, out_ref):
    g = jnp.dot(x_ref[...], w_ref[...], preferred_element_type=jnp.float32) + b_ref[...]
    for h in range(NC):
        out_ref[h] = g[:, h * HW:(h + 1) * HW]


def _combine_body(xsu_ref, s2a_ref, s2b_ref, x_ref):
    for c in range(D // CW):
        h, ph = c // 2, c % 2
        sref = s2a_ref if ph == 0 else s2b_ref
        x_ref[:, c * CW:(c + 1) * CW] = (
            xsu_ref[:, c * CW:(c + 1) * CW]
            + sref[h][:, 0:CW] / (sref[h][:, CW:HW] + 1e-6)
        )


def _sc_edge_body(tab, egt, src, dst, m_hbm, s2a_hbm, s2b_hbm, spill_hbm,
                  src_b, dst_b, esiA, esiB, ediA, ediB, bsi, dsc,
                  gA, gB, egA, egB, bs_b, m_b, csgA, csgB,
                  acc, sem0, sem1, sem2, sem3, sem4, semw):
    cid = lax.axis_index("c")
    sid = lax.axis_index("s")
    ebase = sid * EPW

    es_off = (0 * NC + cid) * N
    ed_off = (1 * NC + cid) * N
    bs_off = (2 * NC + cid) * N

    def zero_acc():
        @pl.loop(0, EB)
        def _(r):
            for g in range(HW // LANES):
                csgA[r, pl.ds(g * LANES, LANES)] = jnp.zeros((LANES,), jnp.float32)

        @pl.loop(0, 15)
        def _(z):
            pltpu.sync_copy(csgA, acc.at[pl.ds(sid * NPW + z * EB, EB)])

        pltpu.sync_copy(csgA.at[pl.ds(0, 25)],
                        acc.at[pl.ds(sid * NPW + 600, 25)])
        plsc.subcore_barrier()

    def writeout_acc(dst_hbm):
        plsc.subcore_barrier()
        @pl.when(sid < NS - 1)
        def _():
            pltpu.sync_copy(acc.at[pl.ds(sid * 640, 640)],
                            dst_hbm.at[cid, pl.ds(sid * 640, 640)])

        @pl.when(sid == NS - 1)
        def _():
            pltpu.sync_copy(acc.at[pl.ds(9600, 400)],
                            dst_hbm.at[cid, pl.ds(9600, 400)])

        plsc.subcore_barrier()

    def build_esi(buf, lt):
        el = lt * EB
        for j in _TAIL:
            buf[pl.ds(j, LANES)] = src_b[pl.ds(el + j, LANES)] + es_off

    def build_edi(buf, lt):
        el = lt * EB
        for j in _TAIL:
            buf[pl.ds(j, LANES)] = dst_b[pl.ds(el + j, LANES)] + ed_off

    def build_bsi(lt):
        el = lt * EB
        for j in _TAIL:
            bsi[pl.ds(j, LANES)] = src_b[pl.ds(el + j, LANES)] + bs_off

    def build_dsc(lt):
        el = lt * EB
        for j in _TAIL:
            dsc[pl.ds(j, LANES)] = dst_b[pl.ds(el + j, LANES)]

    def compute(g, eg):
        @pl.loop(0, EB)
        def _(r):
            for gi in range(HW // LANES):
                sl = pl.ds(gi * LANES, LANES)
                mv = eg[r, sl] + g[r, sl]
                m_b[r, sl] = mv
                sg = 1.0 / (1.0 + jnp.exp(-mv))
                half = csgA if gi < CW // LANES else csgB
                co = (gi % (CW // LANES)) * LANES
                half[r, pl.ds(co, LANES)] = sg * bs_b[r, sl]
                half[r, pl.ds(CW + co, LANES)] = sg

    def fire_eg(eoff, buf, sem):
        pltpu.async_copy(egt.at[cid, pl.ds(ebase + eoff, EB)], buf, sem)

    def fire_m(eoff):
        return pltpu.async_copy(
            m_b, m_hbm.at[pl.ds(ebase + eoff, EB), pl.ds(cid * HW, HW)], semw)

    def drain(descr_src, descr_dst, sem):
        pltpu.make_async_copy(descr_src, descr_dst, sem).wait()

    zero_acc()

    @pl.loop(0, NPAIR)
    def _(pr):
        lt = 2 * (pr % NPPS)
        sb = (pr // NPPS) * STG
        ea = sb + lt * EB
        eb2 = ea + EB

        @pl.when(pr % NPPS == 0)
        def _():
            pltpu.sync_copy(src.at[pl.ds(ebase + sb, STG)], src_b)
            pltpu.sync_copy(dst.at[pl.ds(ebase + sb, STG)], dst_b)
            build_esi(esiA, 0)
            pltpu.async_copy(tab.at[esiA], gA, sem0)
            drain(tab.at[esiA], gA, sem0)
            build_edi(ediA, 0)
            pltpu.async_copy(tab.at[ediA], gA, sem0, add=True)
            fire_eg(ea, egA, sem2)
            build_bsi(0)
            pltpu.async_copy(tab.at[bsi], bs_b, sem4)

        build_esi(esiB, lt + 1)
        pltpu.async_copy(tab.at[esiB], gB, sem1)
        drain(tab.at[ediA], gA, sem0)
        drain(egt.at[cid, pl.ds(ebase + ea, EB)], egA, sem2)
        drain(tab.at[bsi], bs_b, sem4)
        compute(gA, egA)
        wm = fire_m(ea)
        ws = pltpu.async_copy(csgB, spill_hbm.at[cid, pl.ds(ebase + ea, EB)],
                              semw)
        drain(tab.at[esiB], gB, sem1)
        build_edi(ediB, lt + 1)
        pltpu.async_copy(tab.at[ediB], gB, sem1, add=True)
        fire_eg(eb2, egB, sem3)
        build_bsi(lt + 1)
        pltpu.async_copy(tab.at[bsi], bs_b, sem4)
        build_dsc(lt)
        pltpu.sync_copy(csgA, acc.at[dsc], add=True)
        wm.wait()
        ws.wait()

        @pl.when(pr % NPPS < NPPS - 1)
        def _():
            build_esi(esiA, lt + 2)
            pltpu.async_copy(tab.at[esiA], gA, sem0)

        drain(tab.at[ediB], gB, sem1)
        drain(egt.at[cid, pl.ds(ebase + eb2, EB)], egB, sem3)
        drain(tab.at[bsi], bs_b, sem4)
        compute(gB, egB)
        wm2 = fire_m(eb2)
        ws2 = pltpu.async_copy(csgB, spill_hbm.at[cid, pl.ds(ebase + eb2, EB)],
                               semw)

        @pl.when(pr % NPPS < NPPS - 1)
        def _():
            drain(tab.at[esiA], gA, sem0)
            build_edi(ediA, lt + 2)
            pltpu.async_copy(tab.at[ediA], gA, sem0, add=True)
            fire_eg(ea + 2 * EB, egA, sem2)
            build_bsi(lt + 2)
            pltpu.async_copy(tab.at[bsi], bs_b, sem4)

        build_dsc(lt + 1)
        pltpu.sync_copy(csgA, acc.at[dsc], add=True)
        wm2.wait()
        ws2.wait()

    writeout_acc(s2a_hbm)

    zero_acc()

    pltpu.async_copy(spill_hbm.at[cid, pl.ds(ebase, EB)], csgB, sem0)

    @pl.loop(0, NPAIR)
    def _(pr):
        lt = 2 * (pr % NPPS)
        sb = (pr // NPPS) * STG
        e0 = sb + lt * EB

        @pl.when(pr % NPPS == 0)
        def _():
            pltpu.sync_copy(dst.at[pl.ds(ebase + sb, STG)], dst_b)

        drain(spill_hbm.at[cid, pl.ds(ebase + e0, EB)], csgB, sem0)
        pltpu.async_copy(spill_hbm.at[cid, pl.ds(ebase + e0 + EB, EB)],
                         m_b, sem1)
        build_dsc(lt)
        pltpu.sync_copy(csgB, acc.at[dsc], add=True)

        drain(spill_hbm.at[cid, pl.ds(ebase + e0 + EB, EB)], m_b, sem1)

        @pl.when(pr < NPAIR - 1)
        def _():
            pltpu.async_copy(
                spill_hbm.at[cid, pl.ds(ebase + e0 + 2 * EB, EB)], csgB, sem0)

        build_dsc(lt + 1)
        pltpu.sync_copy(m_b, acc.at[dsc], add=True)

    writeout_acc(s2b_hbm)


def _node_proj(node_feats, Wg, bg, W_su, b_su):
    return pl.pallas_call(
        _node_proj_body,
        grid=(N // _NODE_BLK,),
        in_specs=[
            pl.BlockSpec((_NODE_BLK, D), lambda i: (i, 0)),
            pl.BlockSpec((D, 3 * D), lambda i: (0, 0)),
            pl.BlockSpec((1, 3 * D), lambda i: (0, 0)),
            pl.BlockSpec((D, D), lambda i: (0, 0)),
            pl.BlockSpec((1, D), lambda i: (0, 0)),
        ],
        out_specs=[
            pl.BlockSpec((3 * NC, _NODE_BLK, HW), lambda i: (0, i, 0)),
            pl.BlockSpec((_NODE_BLK, D), lambda i: (i, 0)),
        ],
        out_shape=[
            jax.ShapeDtypeStruct((3 * NC, N, HW), jnp.float32),
            jax.ShapeDtypeStruct((N, D), jnp.float32),
        ],
    )(node_feats, Wg, bg, W_su, b_su)


def _edge_proj(edge_feats, W_eg, b_eg):
    return pl.pallas_call(
        _edge_proj_body,
        grid=(E // _EDGE_BLK,),
        in_specs=[
            pl.BlockSpec((_EDGE_BLK, D), lambda i: (i, 0)),
            pl.BlockSpec((D, D), lambda i: (0, 0)),
            pl.BlockSpec((1, D), lambda i: (0, 0)),
        ],
        out_specs=pl.BlockSpec((NC, _EDGE_BLK, HW), lambda i: (0, i, 0)),
        out_shape=jax.ShapeDtypeStruct((NC, E, HW), jnp.float32),
    )(edge_feats, W_eg, b_eg)


def _combine(xsu, s2a, s2b):
    return pl.pallas_call(
        _combine_body,
        grid=(N // _NODE_BLK,),
        in_specs=[
            pl.BlockSpec((_NODE_BLK, D), lambda i: (i, 0)),
            pl.BlockSpec((NC, _NODE_BLK, HW), lambda i: (0, i, 0)),
            pl.BlockSpec((NC, _NODE_BLK, HW), lambda i: (0, i, 0)),
        ],
        out_specs=pl.BlockSpec((_NODE_BLK, D), lambda i: (i, 0)),
        out_shape=jax.ShapeDtypeStruct((N, D), jnp.float32),
    )(xsu, s2a, s2b)


def _sc_edge(tab, egt, src, dst):
    mesh = plsc.VectorSubcoreMesh(core_axis_name="c", subcore_axis_name="s")
    f = pl.kernel(
        _sc_edge_body,
        mesh=mesh,
        out_type=[
            jax.ShapeDtypeStruct((E, D), jnp.float32),
            jax.ShapeDtypeStruct((NC, N, HW), jnp.float32),
            jax.ShapeDtypeStruct((NC, N, HW), jnp.float32),
            jax.ShapeDtypeStruct((NC, E, HW), jnp.float32),
        ],
        scratch_types=[
            pltpu.VMEM((STG,), jnp.int32),
            pltpu.VMEM((STG,), jnp.int32),
            pltpu.VMEM((EB,), jnp.int32),
            pltpu.VMEM((EB,), jnp.int32),
            pltpu.VMEM((EB,), jnp.int32),
            pltpu.VMEM((EB,), jnp.int32),
            pltpu.VMEM((EB,), jnp.int32),
            pltpu.VMEM((EB,), jnp.int32),
            pltpu.VMEM((EB, HW), jnp.float32),
            pltpu.VMEM((EB, HW), jnp.float32),
            pltpu.VMEM((EB, HW), jnp.float32),
            pltpu.VMEM((EB, HW), jnp.float32),
            pltpu.VMEM((EB, HW), jnp.float32),
            pltpu.VMEM((EB, HW), jnp.float32),
            pltpu.VMEM((EB, HW), jnp.float32),
            pltpu.VMEM((EB, HW), jnp.float32),
            pltpu.VMEM_SHARED((N, HW), jnp.float32),
            pltpu.SemaphoreType.DMA,
            pltpu.SemaphoreType.DMA,
            pltpu.SemaphoreType.DMA,
            pltpu.SemaphoreType.DMA,
            pltpu.SemaphoreType.DMA,
            pltpu.SemaphoreType.DMA,
        ],
    )
    return f(tab, egt, src, dst)


@jax.jit
def kernel(node_feats, edge_feats, edge_index, W_sg, b_sg, W_dg, b_dg,
           W_eg, b_eg, W_su, b_su, W_du, b_du):
    src = edge_index[0].astype(jnp.int32)
    dst = edge_index[1].astype(jnp.int32)

    Wg = jnp.concatenate([W_sg, W_dg, W_du], axis=1)
    bg = jnp.concatenate([b_sg, b_dg, b_du]).reshape(1, 3 * D)

    tab, xsu = _node_proj(node_feats, Wg, bg, W_su, b_su.reshape(1, D))
    egt = _edge_proj(edge_feats, W_eg, b_eg.reshape(1, D))

    m, s2a, s2b, _spill = _sc_edge(
        tab.reshape(3 * NC * N, HW), egt, src, dst)
    x = _combine(xsu, s2a, s2b)
    return (x, m)


# speedup vs baseline: 1.0110x; 1.0110x over previous
import jax
import jax.numpy as jnp
from jax import lax
from jax.experimental import pallas as pl
from jax.experimental.pallas import tpu as pltpu
from jax.experimental.pallas import tpu_sc as plsc

N = 10000
E = 160000
D = 256
NC = 2
NS = 16
LANES = 16
HW = 128
CW = 64
EPW = E // NS
EB = 40
STG = 2000
NPPS = STG // (2 * EB)
NPAIR = EPW // (2 * EB)
NPW = N // NS
_TAIL = (0, 16, 24)

_NODE_BLK = 1000
_EDGE_BLK = 2000


def _node_proj_body(x_ref, wg_ref, bg_ref, wsu_ref, bsu_ref, t_ref, xsu_ref):
    x = x_ref[...].astype(jnp.bfloat16)
    g = jnp.dot(x, wg_ref[...], preferred_element_type=jnp.float32) + bg_ref[...]
    for q in range(3):
        for h in range(NC):
            t_ref[q * NC + h] = g[:, q * D + h * HW:q * D + (h + 1) * HW]
    xsu_ref[...] = (
        jnp.dot(x, wsu_ref[...], preferred_element_type=jnp.float32) + bsu_ref[...]
    )


def _edge_proj_body(x_ref, w_ref, b_ref, out_ref):
    g = jnp.dot(x_ref[...].astype(jnp.bfloat16), w_ref[...],
                preferred_element_type=jnp.float32) + b_ref[...]
    for h in range(NC):
        out_ref[h] = g[:, h * HW:(h + 1) * HW]


def _combine_body(xsu_ref, s2a_ref, s2b_ref, x_ref):
    for c in range(D // CW):
        h, ph = c // 2, c % 2
        sref = s2a_ref if ph == 0 else s2b_ref
        x_ref[:, c * CW:(c + 1) * CW] = (
            xsu_ref[:, c * CW:(c + 1) * CW]
            + sref[h][:, 0:CW] / (sref[h][:, CW:HW] + 1e-6)
        )


def _sc_edge_body(tab, egt, src, dst, m_hbm, s2a_hbm, s2b_hbm, spill_hbm,
                  src_b, dst_b, eidxA, eidxB, bsi, dsc,
                  esedA, esedB, bs_b, eg_b, m_b, csgA, csgB,
                  acc, sem0, sem1, sem2, sem3, semw):
    cid = lax.axis_index("c")
    sid = lax.axis_index("s")
    ebase = sid * EPW

    es_off = (0 * NC + cid) * N
    ed_off = (1 * NC + cid) * N
    bs_off = (2 * NC + cid) * N

    def zero_acc():
        @pl.loop(0, EB)
        def _(r):
            for g in range(HW // LANES):
                csgA[r, pl.ds(g * LANES, LANES)] = jnp.zeros((LANES,), jnp.float32)

        @pl.loop(0, 15)
        def _(z):
            pltpu.sync_copy(csgA, acc.at[pl.ds(sid * NPW + z * EB, EB)])

        pltpu.sync_copy(csgA.at[pl.ds(0, 25)],
                        acc.at[pl.ds(sid * NPW + 600, 25)])
        plsc.subcore_barrier()

    def writeout_acc(dst_hbm):
        plsc.subcore_barrier()
        @pl.when(sid < NS - 1)
        def _():
            pltpu.sync_copy(acc.at[pl.ds(sid * 640, 640)],
                            dst_hbm.at[cid, pl.ds(sid * 640, 640)])

        @pl.when(sid == NS - 1)
        def _():
            pltpu.sync_copy(acc.at[pl.ds(9600, 400)],
                            dst_hbm.at[cid, pl.ds(9600, 400)])

        plsc.subcore_barrier()

    def build_idx(ebuf, lt):
        el = lt * EB
        for j in _TAIL:
            ebuf[pl.ds(j, LANES)] = src_b[pl.ds(el + j, LANES)] + es_off
            ebuf[pl.ds(EB + j, LANES)] = dst_b[pl.ds(el + j, LANES)] + ed_off

    def build_bsi(lt):
        el = lt * EB
        for j in _TAIL:
            bsi[pl.ds(j, LANES)] = src_b[pl.ds(el + j, LANES)] + bs_off

    def build_dsc(lt):
        el = lt * EB
        for j in _TAIL:
            dsc[pl.ds(j, LANES)] = dst_b[pl.ds(el + j, LANES)]

    def compute(esed):
        @pl.loop(0, EB)
        def _(r):
            for g in range(HW // LANES):
                sl = pl.ds(g * LANES, LANES)
                mv = eg_b[r, sl] + esed[r, sl] + esed[EB + r, sl]
                m_b[r, sl] = mv
                sg = 1.0 / (1.0 + jnp.exp(-mv))
                half = csgA if g < CW // LANES else csgB
                co = (g % (CW // LANES)) * LANES
                half[r, pl.ds(co, LANES)] = sg * bs_b[r, sl]
                half[r, pl.ds(CW + co, LANES)] = sg

    def fire_eg(eoff):
        pltpu.async_copy(egt.at[cid, pl.ds(ebase + eoff, EB)], eg_b, sem3)

    def fire_m(eoff):
        return pltpu.async_copy(
            m_b, m_hbm.at[pl.ds(ebase + eoff, EB), pl.ds(cid * HW, HW)], semw)

    def drain(descr_src, descr_dst, sem):
        pltpu.make_async_copy(descr_src, descr_dst, sem).wait()

    zero_acc()

    @pl.loop(0, NPAIR)
    def _(pr):
        lt = 2 * (pr % NPPS)
        sb = (pr // NPPS) * STG
        ea = sb + lt * EB
        eb2 = ea + EB

        @pl.when(pr % NPPS == 0)
        def _():
            pltpu.sync_copy(src.at[pl.ds(ebase + sb, STG)], src_b)
            pltpu.sync_copy(dst.at[pl.ds(ebase + sb, STG)], dst_b)
            build_idx(eidxA, 0)
            pltpu.async_copy(tab.at[eidxA], esedA, sem0)
            build_bsi(0)
            pltpu.async_copy(tab.at[bsi], bs_b, sem2)
            fire_eg(ea)

        build_idx(eidxB, lt + 1)
        pltpu.async_copy(tab.at[eidxB], esedB, sem1)
        drain(tab.at[eidxA], esedA, sem0)
        drain(tab.at[bsi], bs_b, sem2)
        drain(egt.at[cid, pl.ds(ebase + ea, EB)], eg_b, sem3)
        compute(esedA)
        wm = fire_m(ea)
        ws = pltpu.async_copy(csgB, spill_hbm.at[cid, pl.ds(ebase + ea, EB)],
                              semw)
        build_bsi(lt + 1)
        pltpu.async_copy(tab.at[bsi], bs_b, sem2)
        fire_eg(eb2)
        build_dsc(lt)
        pltpu.sync_copy(csgA, acc.at[dsc], add=True)
        wm.wait()
        ws.wait()

        @pl.when(pr % NPPS < NPPS - 1)
        def _():
            build_idx(eidxA, lt + 2)
            pltpu.async_copy(tab.at[eidxA], esedA, sem0)

        drain(tab.at[eidxB], esedB, sem1)
        drain(tab.at[bsi], bs_b, sem2)
        drain(egt.at[cid, pl.ds(ebase + eb2, EB)], eg_b, sem3)
        compute(esedB)
        wm2 = fire_m(eb2)
        ws2 = pltpu.async_copy(csgB, spill_hbm.at[cid, pl.ds(ebase + eb2, EB)],
                               semw)

        @pl.when(pr % NPPS < NPPS - 1)
        def _():
            build_bsi(lt + 2)
            pltpu.async_copy(tab.at[bsi], bs_b, sem2)
            fire_eg(ea + 2 * EB)

        build_dsc(lt + 1)
        pltpu.sync_copy(csgA, acc.at[dsc], add=True)
        wm2.wait()
        ws2.wait()

    writeout_acc(s2a_hbm)

    zero_acc()

    pltpu.async_copy(spill_hbm.at[cid, pl.ds(ebase, EB)], csgB, sem0)

    @pl.loop(0, NPAIR)
    def _(pr):
        lt = 2 * (pr % NPPS)
        sb = (pr // NPPS) * STG
        e0 = sb + lt * EB

        @pl.when(pr % NPPS == 0)
        def _():
            pltpu.sync_copy(dst.at[pl.ds(ebase + sb, STG)], dst_b)

        drain(spill_hbm.at[cid, pl.ds(ebase + e0, EB)], csgB, sem0)
        pltpu.async_copy(spill_hbm.at[cid, pl.ds(ebase + e0 + EB, EB)],
                         m_b, sem1)
        build_dsc(lt)
        pltpu.sync_copy(csgB, acc.at[dsc], add=True)

        drain(spill_hbm.at[cid, pl.ds(ebase + e0 + EB, EB)], m_b, sem1)

        @pl.when(pr < NPAIR - 1)
        def _():
            pltpu.async_copy(
                spill_hbm.at[cid, pl.ds(ebase + e0 + 2 * EB, EB)], csgB, sem0)

        build_dsc(lt + 1)
        pltpu.sync_copy(m_b, acc.at[dsc], add=True)

    writeout_acc(s2b_hbm)


def _node_proj(node_feats, Wg, bg, W_su, b_su):
    return pl.pallas_call(
        _node_proj_body,
        grid=(N // _NODE_BLK,),
        in_specs=[
            pl.BlockSpec((_NODE_BLK, D), lambda i: (i, 0)),
            pl.BlockSpec((D, 3 * D), lambda i: (0, 0)),
            pl.BlockSpec((1, 3 * D), lambda i: (0, 0)),
            pl.BlockSpec((D, D), lambda i: (0, 0)),
            pl.BlockSpec((1, D), lambda i: (0, 0)),
        ],
        out_specs=[
            pl.BlockSpec((3 * NC, _NODE_BLK, HW), lambda i: (0, i, 0)),
            pl.BlockSpec((_NODE_BLK, D), lambda i: (i, 0)),
        ],
        out_shape=[
            jax.ShapeDtypeStruct((3 * NC, N, HW), jnp.float32),
            jax.ShapeDtypeStruct((N, D), jnp.float32),
        ],
    )(node_feats, Wg, bg, W_su, b_su)


def _edge_proj(edge_feats, W_eg, b_eg):
    return pl.pallas_call(
        _edge_proj_body,
        grid=(E // _EDGE_BLK,),
        in_specs=[
            pl.BlockSpec((_EDGE_BLK, D), lambda i: (i, 0)),
            pl.BlockSpec((D, D), lambda i: (0, 0)),
            pl.BlockSpec((1, D), lambda i: (0, 0)),
        ],
        out_specs=pl.BlockSpec((NC, _EDGE_BLK, HW), lambda i: (0, i, 0)),
        out_shape=jax.ShapeDtypeStruct((NC, E, HW), jnp.float32),
    )(edge_feats, W_eg, b_eg)


def _combine(xsu, s2a, s2b):
    return pl.pallas_call(
        _combine_body,
        grid=(N // _NODE_BLK,),
        in_specs=[
            pl.BlockSpec((_NODE_BLK, D), lambda i: (i, 0)),
            pl.BlockSpec((NC, _NODE_BLK, HW), lambda i: (0, i, 0)),
            pl.BlockSpec((NC, _NODE_BLK, HW), lambda i: (0, i, 0)),
        ],
        out_specs=pl.BlockSpec((_NODE_BLK, D), lambda i: (i, 0)),
        out_shape=jax.ShapeDtypeStruct((N, D), jnp.float32),
    )(xsu, s2a, s2b)


def _sc_edge(tab, egt, src, dst):
    mesh = plsc.VectorSubcoreMesh(core_axis_name="c", subcore_axis_name="s")
    f = pl.kernel(
        _sc_edge_body,
        mesh=mesh,
        out_type=[
            jax.ShapeDtypeStruct((E, D), jnp.float32),
            jax.ShapeDtypeStruct((NC, N, HW), jnp.float32),
            jax.ShapeDtypeStruct((NC, N, HW), jnp.float32),
            jax.ShapeDtypeStruct((NC, E, HW), jnp.float32),
        ],
        scratch_types=[
            pltpu.VMEM((STG,), jnp.int32),
            pltpu.VMEM((STG,), jnp.int32),
            pltpu.VMEM((2 * EB,), jnp.int32),
            pltpu.VMEM((2 * EB,), jnp.int32),
            pltpu.VMEM((EB,), jnp.int32),
            pltpu.VMEM((EB,), jnp.int32),
            pltpu.VMEM((2 * EB, HW), jnp.float32),
            pltpu.VMEM((2 * EB, HW), jnp.float32),
            pltpu.VMEM((EB, HW), jnp.float32),
            pltpu.VMEM((EB, HW), jnp.float32),
            pltpu.VMEM((EB, HW), jnp.float32),
            pltpu.VMEM((EB, HW), jnp.float32),
            pltpu.VMEM((EB, HW), jnp.float32),
            pltpu.VMEM_SHARED((N, HW), jnp.float32),
            pltpu.SemaphoreType.DMA,
            pltpu.SemaphoreType.DMA,
            pltpu.SemaphoreType.DMA,
            pltpu.SemaphoreType.DMA,
            pltpu.SemaphoreType.DMA,
        ],
    )
    return f(tab, egt, src, dst)


@jax.jit
def kernel(node_feats, edge_feats, edge_index, W_sg, b_sg, W_dg, b_dg,
           W_eg, b_eg, W_su, b_su, W_du, b_du):
    src = edge_index[0].astype(jnp.int32)
    dst = edge_index[1].astype(jnp.int32)

    Wg = jnp.concatenate([W_sg, W_dg, W_du], axis=1).astype(jnp.bfloat16)
    bg = jnp.concatenate([b_sg, b_dg, b_du]).reshape(1, 3 * D)

    tab, xsu = _node_proj(node_feats, Wg, bg,
                          W_su.astype(jnp.bfloat16), b_su.reshape(1, D))
    egt = _edge_proj(edge_feats, W_eg.astype(jnp.bfloat16),
                     b_eg.reshape(1, D))

    m, s2a, s2b, _spill = _sc_edge(
        tab.reshape(3 * NC * N, HW), egt, src, dst)
    x = _combine(xsu, s2a, s2b)
    return (x, m)


# speedup vs baseline: 1.1049x; 1.0928x over previous
import jax
import jax.numpy as jnp
from jax import lax
from jax.experimental import pallas as pl
from jax.experimental.pallas import tpu as pltpu
from jax.experimental.pallas import tpu_sc as plsc

N = 10000
E = 160000
D = 256
NC = 2
NS = 16
LANES = 16
HW = 128
CW = 64
EPW = E // NS
EB = 40
STG = 2000
NPPS = STG // (2 * EB)
NPAIR = EPW // (2 * EB)
NPW = N // NS
_TAIL = (0, 16, 24)

_NODE_BLK = 1000
_EDGE_BLK = 2000


def _node_proj_body(x_ref, wg_ref, bg_ref, wsu_ref, bsu_ref, t_ref, xsu_ref):
    x = x_ref[...]
    g = jnp.dot(x, wg_ref[...], preferred_element_type=jnp.float32) + bg_ref[...]
    for q in range(3):
        for h in range(NC):
            t_ref[q * NC + h] = g[:, q * D + h * HW:q * D + (h + 1) * HW]
    xsu_ref[...] = (
        jnp.dot(x, wsu_ref[...], preferred_element_type=jnp.float32) + bsu_ref[...]
    )


def _edge_proj_body(x_ref, w_ref, b_ref, out_ref):
    g = jnp.dot(x_ref[...], w_ref[...], preferred_element_type=jnp.float32) + b_ref[...]
    for h in range(NC):
        out_ref[h] = g[:, h * HW:(h + 1) * HW]


def _combine_body(xsu_ref, s2a_ref, s2b_ref, x_ref):
    for c in range(D // CW):
        h, ph = c // 2, c % 2
        sref = s2a_ref if ph == 0 else s2b_ref
        x_ref[:, c * CW:(c + 1) * CW] = (
            xsu_ref[:, c * CW:(c + 1) * CW]
            + sref[h][:, 0:CW] / (sref[h][:, CW:HW] + 1e-6)
        )


def _sc_edge_body(tab, egt, src, dst, m_hbm, s2a_hbm, s2b_hbm, spill_hbm,
                  src_b, dst_b, eidxA, eidxB, bsi, dsc,
                  esedA, esedB, bs_b, eg_b, m_b, csgA, csgB,
                  acc, sem0, sem1, sem2, sem3, semw):
    cid = lax.axis_index("c")
    sid = lax.axis_index("s")
    ebase = sid * EPW

    es_off = (0 * NC + cid) * N
    ed_off = (1 * NC + cid) * N
    bs_off = (2 * NC + cid) * N

    def zero_acc():
        @pl.loop(0, EB)
        def _(r):
            for g in range(HW // LANES):
                csgA[r, pl.ds(g * LANES, LANES)] = jnp.zeros((LANES,), jnp.float32)

        @pl.loop(0, 15)
        def _(z):
            pltpu.sync_copy(csgA, acc.at[pl.ds(sid * NPW + z * EB, EB)])

        pltpu.sync_copy(csgA.at[pl.ds(0, 25)],
                        acc.at[pl.ds(sid * NPW + 600, 25)])
        plsc.subcore_barrier()

    def writeout_acc(dst_hbm):
        plsc.subcore_barrier()
        @pl.when(sid < NS - 1)
        def _():
            pltpu.sync_copy(acc.at[pl.ds(sid * 640, 640)],
                            dst_hbm.at[cid, pl.ds(sid * 640, 640)])

        @pl.when(sid == NS - 1)
        def _():
            pltpu.sync_copy(acc.at[pl.ds(9600, 400)],
                            dst_hbm.at[cid, pl.ds(9600, 400)])

        plsc.subcore_barrier()

    def build_idx(ebuf, lt):
        el = lt * EB
        for j in _TAIL:
            ebuf[pl.ds(j, LANES)] = src_b[pl.ds(el + j, LANES)] + es_off
            ebuf[pl.ds(EB + j, LANES)] = dst_b[pl.ds(el + j, LANES)] + ed_off

    def build_bsi(lt):
        el = lt * EB
        for j in _TAIL:
            bsi[pl.ds(j, LANES)] = src_b[pl.ds(el + j, LANES)] + bs_off

    def build_dsc(lt):
        el = lt * EB
        for j in _TAIL:
            dsc[pl.ds(j, LANES)] = dst_b[pl.ds(el + j, LANES)]

    def compute(esed):
        @pl.loop(0, EB)
        def _(r):
            for g in range(HW // LANES):
                sl = pl.ds(g * LANES, LANES)
                mv = eg_b[r, sl] + esed[r, sl] + esed[EB + r, sl]
                m_b[r, sl] = mv
                sg = 1.0 / (1.0 + jnp.exp(-mv))
                half = csgA if g < CW // LANES else csgB
                co = (g % (CW // LANES)) * LANES
                half[r, pl.ds(co, LANES)] = sg * bs_b[r, sl]
                half[r, pl.ds(CW + co, LANES)] = sg

    def fire_eg(eoff):
        pltpu.async_copy(egt.at[cid, pl.ds(ebase + eoff, EB)], eg_b, sem3)

    def fire_m(eoff):
        return pltpu.async_copy(
            m_b, m_hbm.at[pl.ds(ebase + eoff, EB), pl.ds(cid * HW, HW)], semw)

    def drain(descr_src, descr_dst, sem):
        pltpu.make_async_copy(descr_src, descr_dst, sem).wait()

    zero_acc()

    @pl.loop(0, NPAIR)
    def _(pr):
        lt = 2 * (pr % NPPS)
        sb = (pr // NPPS) * STG
        ea = sb + lt * EB
        eb2 = ea + EB

        @pl.when(pr % NPPS == 0)
        def _():
            pltpu.sync_copy(src.at[pl.ds(ebase + sb, STG)], src_b)
            pltpu.sync_copy(dst.at[pl.ds(ebase + sb, STG)], dst_b)
            build_idx(eidxA, 0)
            pltpu.async_copy(tab.at[eidxA], esedA, sem0)
            build_bsi(0)
            pltpu.async_copy(tab.at[bsi], bs_b, sem2)
            fire_eg(ea)

        build_idx(eidxB, lt + 1)
        pltpu.async_copy(tab.at[eidxB], esedB, sem1)
        drain(tab.at[eidxA], esedA, sem0)
        drain(tab.at[bsi], bs_b, sem2)
        drain(egt.at[cid, pl.ds(ebase + ea, EB)], eg_b, sem3)
        compute(esedA)
        wm = fire_m(ea)
        ws = pltpu.async_copy(csgB, spill_hbm.at[cid, pl.ds(ebase + ea, EB)],
                              semw)
        build_bsi(lt + 1)
        pltpu.async_copy(tab.at[bsi], bs_b, sem2)
        fire_eg(eb2)
        build_dsc(lt)
        pltpu.sync_copy(csgA, acc.at[dsc], add=True)
        wm.wait()
        ws.wait()

        @pl.when(pr % NPPS < NPPS - 1)
        def _():
            build_idx(eidxA, lt + 2)
            pltpu.async_copy(tab.at[eidxA], esedA, sem0)

        drain(tab.at[eidxB], esedB, sem1)
        drain(tab.at[bsi], bs_b, sem2)
        drain(egt.at[cid, pl.ds(ebase + eb2, EB)], eg_b, sem3)
        compute(esedB)
        wm2 = fire_m(eb2)
        ws2 = pltpu.async_copy(csgB, spill_hbm.at[cid, pl.ds(ebase + eb2, EB)],
                               semw)

        @pl.when(pr % NPPS < NPPS - 1)
        def _():
            build_bsi(lt + 2)
            pltpu.async_copy(tab.at[bsi], bs_b, sem2)
            fire_eg(ea + 2 * EB)

        build_dsc(lt + 1)
        pltpu.sync_copy(csgA, acc.at[dsc], add=True)
        wm2.wait()
        ws2.wait()

    writeout_acc(s2a_hbm)

    B2 = 2 * EB
    NB2 = EPW // B2
    BPS = STG // B2

    def build_dsc80(buf, t):
        el = (t % BPS) * B2
        for j in range(0, B2, LANES):
            buf[pl.ds(j, LANES)] = dst_b[pl.ds(el + j, LANES)]

    def reload_if_stage_top(t):
        @pl.when(t % BPS == 0)
        def _():
            pltpu.sync_copy(
                dst.at[pl.ds(ebase + (t // BPS) * STG, STG)], dst_b)

    zero_acc()

    pltpu.async_copy(spill_hbm.at[cid, pl.ds(ebase, B2)], esedA, sem0)

    @pl.loop(0, NB2 // 2)
    def _(pr):
        t0 = 2 * pr
        e0 = t0 * B2

        reload_if_stage_top(t0)
        drain(spill_hbm.at[cid, pl.ds(ebase + e0, B2)], esedA, sem0)
        pltpu.async_copy(spill_hbm.at[cid, pl.ds(ebase + e0 + B2, B2)],
                         esedB, sem1)
        build_dsc80(eidxA, t0)
        pltpu.sync_copy(esedA, acc.at[eidxA], add=True)

        reload_if_stage_top(t0 + 1)
        drain(spill_hbm.at[cid, pl.ds(ebase + e0 + B2, B2)], esedB, sem1)
        pltpu.async_copy(spill_hbm.at[cid, pl.ds(ebase + e0 + 2 * B2, B2)],
                         esedA, sem0)
        build_dsc80(eidxB, t0 + 1)
        pltpu.sync_copy(esedB, acc.at[eidxB], add=True)

    drain(spill_hbm.at[cid, pl.ds(ebase + (NB2 - 1) * B2, B2)], esedA, sem0)
    build_dsc80(eidxA, NB2 - 1)
    pltpu.sync_copy(esedA, acc.at[eidxA], add=True)

    writeout_acc(s2b_hbm)


def _node_proj(node_feats, Wg, bg, W_su, b_su):
    return pl.pallas_call(
        _node_proj_body,
        grid=(N // _NODE_BLK,),
        in_specs=[
            pl.BlockSpec((_NODE_BLK, D), lambda i: (i, 0)),
            pl.BlockSpec((D, 3 * D), lambda i: (0, 0)),
            pl.BlockSpec((1, 3 * D), lambda i: (0, 0)),
            pl.BlockSpec((D, D), lambda i: (0, 0)),
            pl.BlockSpec((1, D), lambda i: (0, 0)),
        ],
        out_specs=[
            pl.BlockSpec((3 * NC, _NODE_BLK, HW), lambda i: (0, i, 0)),
            pl.BlockSpec((_NODE_BLK, D), lambda i: (i, 0)),
        ],
        out_shape=[
            jax.ShapeDtypeStruct((3 * NC, N, HW), jnp.float32),
            jax.ShapeDtypeStruct((N, D), jnp.float32),
        ],
    )(node_feats, Wg, bg, W_su, b_su)


def _edge_proj(edge_feats, W_eg, b_eg):
    return pl.pallas_call(
        _edge_proj_body,
        grid=(E // _EDGE_BLK,),
        in_specs=[
            pl.BlockSpec((_EDGE_BLK, D), lambda i: (i, 0)),
            pl.BlockSpec((D, D), lambda i: (0, 0)),
            pl.BlockSpec((1, D), lambda i: (0, 0)),
        ],
        out_specs=pl.BlockSpec((NC, _EDGE_BLK, HW), lambda i: (0, i, 0)),
        out_shape=jax.ShapeDtypeStruct((NC, E, HW), jnp.float32),
    )(edge_feats, W_eg, b_eg)


def _combine(xsu, s2a, s2b):
    return pl.pallas_call(
        _combine_body,
        grid=(N // _NODE_BLK,),
        in_specs=[
            pl.BlockSpec((_NODE_BLK, D), lambda i: (i, 0)),
            pl.BlockSpec((NC, _NODE_BLK, HW), lambda i: (0, i, 0)),
            pl.BlockSpec((NC, _NODE_BLK, HW), lambda i: (0, i, 0)),
        ],
        out_specs=pl.BlockSpec((_NODE_BLK, D), lambda i: (i, 0)),
        out_shape=jax.ShapeDtypeStruct((N, D), jnp.float32),
    )(xsu, s2a, s2b)


def _sc_edge(tab, egt, src, dst):
    mesh = plsc.VectorSubcoreMesh(core_axis_name="c", subcore_axis_name="s")
    f = pl.kernel(
        _sc_edge_body,
        mesh=mesh,
        out_type=[
            jax.ShapeDtypeStruct((E, D), jnp.float32),
            jax.ShapeDtypeStruct((NC, N, HW), jnp.float32),
            jax.ShapeDtypeStruct((NC, N, HW), jnp.float32),
            jax.ShapeDtypeStruct((NC, E, HW), jnp.float32),
        ],
        scratch_types=[
            pltpu.VMEM((STG,), jnp.int32),
            pltpu.VMEM((STG,), jnp.int32),
            pltpu.VMEM((2 * EB,), jnp.int32),
            pltpu.VMEM((2 * EB,), jnp.int32),
            pltpu.VMEM((EB,), jnp.int32),
            pltpu.VMEM((EB,), jnp.int32),
            pltpu.VMEM((2 * EB, HW), jnp.float32),
            pltpu.VMEM((2 * EB, HW), jnp.float32),
            pltpu.VMEM((EB, HW), jnp.float32),
            pltpu.VMEM((EB, HW), jnp.float32),
            pltpu.VMEM((EB, HW), jnp.float32),
            pltpu.VMEM((EB, HW), jnp.float32),
            pltpu.VMEM((EB, HW), jnp.float32),
            pltpu.VMEM_SHARED((N, HW), jnp.float32),
            pltpu.SemaphoreType.DMA,
            pltpu.SemaphoreType.DMA,
            pltpu.SemaphoreType.DMA,
            pltpu.SemaphoreType.DMA,
            pltpu.SemaphoreType.DMA,
        ],
    )
    return f(tab, egt, src, dst)


@jax.jit
def kernel(node_feats, edge_feats, edge_index, W_sg, b_sg, W_dg, b_dg,
           W_eg, b_eg, W_su, b_su, W_du, b_du):
    src = edge_index[0].astype(jnp.int32)
    dst = edge_index[1].astype(jnp.int32)

    Wg = jnp.concatenate([W_sg, W_dg, W_du], axis=1)
    bg = jnp.concatenate([b_sg, b_dg, b_du]).reshape(1, 3 * D)

    tab, xsu = _node_proj(node_feats, Wg, bg, W_su, b_su.reshape(1, D))
    egt = _edge_proj(edge_feats, W_eg, b_eg.reshape(1, D))

    m, s2a, s2b, _spill = _sc_edge(
        tab.reshape(3 * NC * N, HW), egt, src, dst)
    x = _combine(xsu, s2a, s2b)
    return (x, m)


# speedup vs baseline: 1.1084x; 1.0032x over previous
import jax
import jax.numpy as jnp
from jax import lax
from jax.experimental import pallas as pl
from jax.experimental.pallas import tpu as pltpu
from jax.experimental.pallas import tpu_sc as plsc

N = 10000
E = 160000
D = 256
NC = 2
NS = 16
LANES = 16
HW = 128
CW = 64
EPW = E // NS
EB = 40
STG = 2000
NPPS = STG // (2 * EB)
NPAIR = EPW // (2 * EB)
NPW = N // NS
_TAIL = (0, 16, 24)

_NODE_BLK = 1000
_EDGE_BLK = 2000


def _node_proj_body(x_ref, wsg_ref, bsg_ref, wdg_ref, bdg_ref, wdu_ref,
                    bdu_ref, wsu_ref, bsu_ref, t_ref, xsu_ref):
    x = x_ref[...]
    for q, (w, b) in enumerate(((wsg_ref, bsg_ref), (wdg_ref, bdg_ref),
                                (wdu_ref, bdu_ref))):
        g = jnp.dot(x, w[...], preferred_element_type=jnp.float32) + b[...]
        for h in range(NC):
            t_ref[q * NC + h] = g[:, h * HW:(h + 1) * HW]
    xsu_ref[...] = (
        jnp.dot(x, wsu_ref[...], preferred_element_type=jnp.float32) + bsu_ref[...]
    )


def _edge_proj_body(x_ref, w_ref, b_ref, out_ref):
    g = jnp.dot(x_ref[...], w_ref[...], preferred_element_type=jnp.float32) + b_ref[...]
    for h in range(NC):
        out_ref[h] = g[:, h * HW:(h + 1) * HW]


def _combine_body(xsu_ref, s2a_ref, s2b_ref, x_ref):
    for c in range(D // CW):
        h, ph = c // 2, c % 2
        sref = s2a_ref if ph == 0 else s2b_ref
        x_ref[:, c * CW:(c + 1) * CW] = (
            xsu_ref[:, c * CW:(c + 1) * CW]
            + sref[h][:, 0:CW] / (sref[h][:, CW:HW] + 1e-6)
        )


def _sc_edge_body(tab, egt, src, dst, m_hbm, s2a_hbm, s2b_hbm, spill_hbm,
                  src_b, dst_b, eidxA, eidxB, bsi, dsc,
                  esedA, esedB, bs_b, eg_b, m_b, csgA, csgB,
                  acc, sem0, sem1, sem2, sem3, semw):
    cid = lax.axis_index("c")
    sid = lax.axis_index("s")
    ebase = sid * EPW

    es_off = (0 * NC + cid) * N
    ed_off = (1 * NC + cid) * N
    bs_off = (2 * NC + cid) * N

    def zero_acc():
        @pl.loop(0, EB)
        def _(r):
            for g in range(HW // LANES):
                csgA[r, pl.ds(g * LANES, LANES)] = jnp.zeros((LANES,), jnp.float32)

        @pl.loop(0, 15)
        def _(z):
            pltpu.sync_copy(csgA, acc.at[pl.ds(sid * NPW + z * EB, EB)])

        pltpu.sync_copy(csgA.at[pl.ds(0, 25)],
                        acc.at[pl.ds(sid * NPW + 600, 25)])
        plsc.subcore_barrier()

    def writeout_acc(dst_hbm):
        plsc.subcore_barrier()
        @pl.when(sid < NS - 1)
        def _():
            pltpu.sync_copy(acc.at[pl.ds(sid * 640, 640)],
                            dst_hbm.at[cid, pl.ds(sid * 640, 640)])

        @pl.when(sid == NS - 1)
        def _():
            pltpu.sync_copy(acc.at[pl.ds(9600, 400)],
                            dst_hbm.at[cid, pl.ds(9600, 400)])

        plsc.subcore_barrier()

    def build_idx(ebuf, lt):
        el = lt * EB
        for j in _TAIL:
            ebuf[pl.ds(j, LANES)] = src_b[pl.ds(el + j, LANES)] + es_off
            ebuf[pl.ds(EB + j, LANES)] = dst_b[pl.ds(el + j, LANES)] + ed_off

    def build_bsi(lt):
        el = lt * EB
        for j in _TAIL:
            bsi[pl.ds(j, LANES)] = src_b[pl.ds(el + j, LANES)] + bs_off

    def build_dsc(lt):
        el = lt * EB
        for j in _TAIL:
            dsc[pl.ds(j, LANES)] = dst_b[pl.ds(el + j, LANES)]

    def compute(esed):
        @pl.loop(0, EB)
        def _(r):
            for g in range(HW // LANES):
                sl = pl.ds(g * LANES, LANES)
                mv = eg_b[r, sl] + esed[r, sl] + esed[EB + r, sl]
                m_b[r, sl] = mv
                sg = 1.0 / (1.0 + jnp.exp(-mv))
                half = csgA if g < CW // LANES else csgB
                co = (g % (CW // LANES)) * LANES
                half[r, pl.ds(co, LANES)] = sg * bs_b[r, sl]
                half[r, pl.ds(CW + co, LANES)] = sg

    def fire_eg(eoff):
        pltpu.async_copy(egt.at[cid, pl.ds(ebase + eoff, EB)], eg_b, sem3)

    def fire_m(eoff):
        return pltpu.async_copy(
            m_b, m_hbm.at[pl.ds(ebase + eoff, EB), pl.ds(cid * HW, HW)], semw)

    def drain(descr_src, descr_dst, sem):
        pltpu.make_async_copy(descr_src, descr_dst, sem).wait()

    zero_acc()

    @pl.loop(0, NPAIR)
    def _(pr):
        lt = 2 * (pr % NPPS)
        sb = (pr // NPPS) * STG
        ea = sb + lt * EB
        eb2 = ea + EB

        @pl.when(pr % NPPS == 0)
        def _():
            pltpu.sync_copy(src.at[pl.ds(ebase + sb, STG)], src_b)
            pltpu.sync_copy(dst.at[pl.ds(ebase + sb, STG)], dst_b)
            build_idx(eidxA, 0)
            pltpu.async_copy(tab.at[eidxA], esedA, sem0)
            build_bsi(0)
            pltpu.async_copy(tab.at[bsi], bs_b, sem2)
            fire_eg(ea)

        build_idx(eidxB, lt + 1)
        pltpu.async_copy(tab.at[eidxB], esedB, sem1)
        drain(tab.at[eidxA], esedA, sem0)
        drain(tab.at[bsi], bs_b, sem2)
        drain(egt.at[cid, pl.ds(ebase + ea, EB)], eg_b, sem3)
        compute(esedA)
        wm = fire_m(ea)
        ws = pltpu.async_copy(csgB, spill_hbm.at[cid, pl.ds(ebase + ea, EB)],
                              semw)
        build_bsi(lt + 1)
        pltpu.async_copy(tab.at[bsi], bs_b, sem2)
        fire_eg(eb2)
        build_dsc(lt)
        pltpu.sync_copy(csgA, acc.at[dsc], add=True)
        wm.wait()
        ws.wait()

        @pl.when(pr % NPPS < NPPS - 1)
        def _():
            build_idx(eidxA, lt + 2)
            pltpu.async_copy(tab.at[eidxA], esedA, sem0)

        drain(tab.at[eidxB], esedB, sem1)
        drain(tab.at[bsi], bs_b, sem2)
        drain(egt.at[cid, pl.ds(ebase + eb2, EB)], eg_b, sem3)
        compute(esedB)
        wm2 = fire_m(eb2)
        ws2 = pltpu.async_copy(csgB, spill_hbm.at[cid, pl.ds(ebase + eb2, EB)],
                               semw)

        @pl.when(pr % NPPS < NPPS - 1)
        def _():
            build_bsi(lt + 2)
            pltpu.async_copy(tab.at[bsi], bs_b, sem2)
            fire_eg(ea + 2 * EB)

        build_dsc(lt + 1)
        pltpu.sync_copy(csgA, acc.at[dsc], add=True)
        wm2.wait()
        ws2.wait()

    writeout_acc(s2a_hbm)

    B2 = 2 * EB
    NB2 = EPW // B2
    BPS = STG // B2

    def build_dsc80(buf, t):
        el = (t % BPS) * B2
        for j in range(0, B2, LANES):
            buf[pl.ds(j, LANES)] = dst_b[pl.ds(el + j, LANES)]

    def reload_if_stage_top(t):
        @pl.when(t % BPS == 0)
        def _():
            pltpu.sync_copy(
                dst.at[pl.ds(ebase + (t // BPS) * STG, STG)], dst_b)

    zero_acc()

    pltpu.async_copy(spill_hbm.at[cid, pl.ds(ebase, B2)], esedA, sem0)

    @pl.loop(0, NB2 // 2)
    def _(pr):
        t0 = 2 * pr
        e0 = t0 * B2

        reload_if_stage_top(t0)
        drain(spill_hbm.at[cid, pl.ds(ebase + e0, B2)], esedA, sem0)
        pltpu.async_copy(spill_hbm.at[cid, pl.ds(ebase + e0 + B2, B2)],
                         esedB, sem1)
        build_dsc80(eidxA, t0)
        pltpu.sync_copy(esedA, acc.at[eidxA], add=True)

        reload_if_stage_top(t0 + 1)
        drain(spill_hbm.at[cid, pl.ds(ebase + e0 + B2, B2)], esedB, sem1)
        pltpu.async_copy(spill_hbm.at[cid, pl.ds(ebase + e0 + 2 * B2, B2)],
                         esedA, sem0)
        build_dsc80(eidxB, t0 + 1)
        pltpu.sync_copy(esedB, acc.at[eidxB], add=True)

    drain(spill_hbm.at[cid, pl.ds(ebase + (NB2 - 1) * B2, B2)], esedA, sem0)
    build_dsc80(eidxA, NB2 - 1)
    pltpu.sync_copy(esedA, acc.at[eidxA], add=True)

    writeout_acc(s2b_hbm)


def _node_proj(node_feats, W_sg, b_sg, W_dg, b_dg, W_du, b_du, W_su, b_su):
    wspec = pl.BlockSpec((D, D), lambda i: (0, 0))
    bspec = pl.BlockSpec((1, D), lambda i: (0, 0))
    return pl.pallas_call(
        _node_proj_body,
        grid=(N // _NODE_BLK,),
        in_specs=[
            pl.BlockSpec((_NODE_BLK, D), lambda i: (i, 0)),
            wspec, bspec, wspec, bspec, wspec, bspec, wspec, bspec,
        ],
        out_specs=[
            pl.BlockSpec((3 * NC, _NODE_BLK, HW), lambda i: (0, i, 0)),
            pl.BlockSpec((_NODE_BLK, D), lambda i: (i, 0)),
        ],
        out_shape=[
            jax.ShapeDtypeStruct((3 * NC, N, HW), jnp.float32),
            jax.ShapeDtypeStruct((N, D), jnp.float32),
        ],
    )(node_feats, W_sg, b_sg, W_dg, b_dg, W_du, b_du, W_su, b_su)


def _edge_proj(edge_feats, W_eg, b_eg):
    return pl.pallas_call(
        _edge_proj_body,
        grid=(E // _EDGE_BLK,),
        in_specs=[
            pl.BlockSpec((_EDGE_BLK, D), lambda i: (i, 0)),
            pl.BlockSpec((D, D), lambda i: (0, 0)),
            pl.BlockSpec((1, D), lambda i: (0, 0)),
        ],
        out_specs=pl.BlockSpec((NC, _EDGE_BLK, HW), lambda i: (0, i, 0)),
        out_shape=jax.ShapeDtypeStruct((NC, E, HW), jnp.float32),
    )(edge_feats, W_eg, b_eg)


def _combine(xsu, s2a, s2b):
    return pl.pallas_call(
        _combine_body,
        grid=(N // _NODE_BLK,),
        in_specs=[
            pl.BlockSpec((_NODE_BLK, D), lambda i: (i, 0)),
            pl.BlockSpec((NC, _NODE_BLK, HW), lambda i: (0, i, 0)),
            pl.BlockSpec((NC, _NODE_BLK, HW), lambda i: (0, i, 0)),
        ],
        out_specs=pl.BlockSpec((_NODE_BLK, D), lambda i: (i, 0)),
        out_shape=jax.ShapeDtypeStruct((N, D), jnp.float32),
    )(xsu, s2a, s2b)


def _sc_edge(tab, egt, src, dst):
    mesh = plsc.VectorSubcoreMesh(core_axis_name="c", subcore_axis_name="s")
    f = pl.kernel(
        _sc_edge_body,
        mesh=mesh,
        out_type=[
            jax.ShapeDtypeStruct((E, D), jnp.float32),
            jax.ShapeDtypeStruct((NC, N, HW), jnp.float32),
            jax.ShapeDtypeStruct((NC, N, HW), jnp.float32),
            jax.ShapeDtypeStruct((NC, E, HW), jnp.float32),
        ],
        scratch_types=[
            pltpu.VMEM((STG,), jnp.int32),
            pltpu.VMEM((STG,), jnp.int32),
            pltpu.VMEM((2 * EB,), jnp.int32),
            pltpu.VMEM((2 * EB,), jnp.int32),
            pltpu.VMEM((EB,), jnp.int32),
            pltpu.VMEM((EB,), jnp.int32),
            pltpu.VMEM((2 * EB, HW), jnp.float32),
            pltpu.VMEM((2 * EB, HW), jnp.float32),
            pltpu.VMEM((EB, HW), jnp.float32),
            pltpu.VMEM((EB, HW), jnp.float32),
            pltpu.VMEM((EB, HW), jnp.float32),
            pltpu.VMEM((EB, HW), jnp.float32),
            pltpu.VMEM((EB, HW), jnp.float32),
            pltpu.VMEM_SHARED((N, HW), jnp.float32),
            pltpu.SemaphoreType.DMA,
            pltpu.SemaphoreType.DMA,
            pltpu.SemaphoreType.DMA,
            pltpu.SemaphoreType.DMA,
            pltpu.SemaphoreType.DMA,
        ],
    )
    return f(tab, egt, src, dst)


@jax.jit
def kernel(node_feats, edge_feats, edge_index, W_sg, b_sg, W_dg, b_dg,
           W_eg, b_eg, W_su, b_su, W_du, b_du):
    src = edge_index[0].astype(jnp.int32)
    dst = edge_index[1].astype(jnp.int32)

    tab, xsu = _node_proj(node_feats, W_sg, b_sg.reshape(1, D),
                          W_dg, b_dg.reshape(1, D), W_du, b_du.reshape(1, D),
                          W_su, b_su.reshape(1, D))
    egt = _edge_proj(edge_feats, W_eg, b_eg.reshape(1, D))

    m, s2a, s2b, _spill = _sc_edge(
        tab.reshape(3 * NC * N, HW), egt, src, dst)
    x = _combine(xsu, s2a, s2b)
    return (x, m)


# speedup vs baseline: 1.1353x; 1.0242x over previous
import jax
import jax.numpy as jnp
from jax import lax
from jax.experimental import pallas as pl
from jax.experimental.pallas import tpu as pltpu
from jax.experimental.pallas import tpu_sc as plsc

N = 10000
E = 160000
D = 256
NC = 2
NS = 16
LANES = 16
HW = 128
CW = 64
EPW = E // NS
EB = 40
STG = 2000
NPPS = STG // (2 * EB)
NPAIR = EPW // (2 * EB)
NPW = N // NS
_TAIL = (0, 16, 24)

_NODE_BLK = 2000
_EDGE_BLK = 4000


def _node_proj_body(x_ref, wsg_ref, bsg_ref, wdg_ref, bdg_ref, wdu_ref,
                    bdu_ref, wsu_ref, bsu_ref, t_ref, xsu_ref):
    x = x_ref[...]
    for q, (w, b) in enumerate(((wsg_ref, bsg_ref), (wdg_ref, bdg_ref),
                                (wdu_ref, bdu_ref))):
        g = jnp.dot(x, w[...], preferred_element_type=jnp.float32) + b[...]
        for h in range(NC):
            t_ref[q * NC + h] = g[:, h * HW:(h + 1) * HW]
    xsu_ref[...] = (
        jnp.dot(x, wsu_ref[...], preferred_element_type=jnp.float32) + bsu_ref[...]
    )


def _edge_proj_body(x_ref, w_ref, b_ref, out_ref):
    g = jnp.dot(x_ref[...], w_ref[...], preferred_element_type=jnp.float32) + b_ref[...]
    for h in range(NC):
        out_ref[h] = g[:, h * HW:(h + 1) * HW]


def _combine_body(xsu_ref, s2a_ref, s2b_ref, x_ref):
    for c in range(D // CW):
        h, ph = c // 2, c % 2
        sref = s2a_ref if ph == 0 else s2b_ref
        x_ref[:, c * CW:(c + 1) * CW] = (
            xsu_ref[:, c * CW:(c + 1) * CW]
            + sref[h][:, 0:CW] / (sref[h][:, CW:HW] + 1e-6)
        )


def _sc_edge_body(tab, egt, src, dst, m_hbm, s2a_hbm, s2b_hbm, spill_hbm,
                  src_b, dst_b, eidxA, eidxB, bsi, dsc,
                  esedA, esedB, bs_b, eg_b, m_b, csgA, csgB,
                  acc, sem0, sem1, sem2, sem3, semw):
    cid = lax.axis_index("c")
    sid = lax.axis_index("s")
    ebase = sid * EPW

    es_off = (0 * NC + cid) * N
    ed_off = (1 * NC + cid) * N
    bs_off = (2 * NC + cid) * N

    def zero_acc():
        @pl.loop(0, EB)
        def _(r):
            for g in range(HW // LANES):
                csgA[r, pl.ds(g * LANES, LANES)] = jnp.zeros((LANES,), jnp.float32)

        @pl.loop(0, 15)
        def _(z):
            pltpu.sync_copy(csgA, acc.at[pl.ds(sid * NPW + z * EB, EB)])

        pltpu.sync_copy(csgA.at[pl.ds(0, 25)],
                        acc.at[pl.ds(sid * NPW + 600, 25)])
        plsc.subcore_barrier()

    def writeout_acc(dst_hbm):
        plsc.subcore_barrier()
        @pl.when(sid < NS - 1)
        def _():
            pltpu.sync_copy(acc.at[pl.ds(sid * 640, 640)],
                            dst_hbm.at[cid, pl.ds(sid * 640, 640)])

        @pl.when(sid == NS - 1)
        def _():
            pltpu.sync_copy(acc.at[pl.ds(9600, 400)],
                            dst_hbm.at[cid, pl.ds(9600, 400)])

        plsc.subcore_barrier()

    def build_idx(ebuf, lt):
        el = lt * EB
        for j in _TAIL:
            ebuf[pl.ds(j, LANES)] = src_b[pl.ds(el + j, LANES)] + es_off
            ebuf[pl.ds(EB + j, LANES)] = dst_b[pl.ds(el + j, LANES)] + ed_off

    def build_bsi(lt):
        el = lt * EB
        for j in _TAIL:
            bsi[pl.ds(j, LANES)] = src_b[pl.ds(el + j, LANES)] + bs_off

    def build_dsc(lt):
        el = lt * EB
        for j in _TAIL:
            dsc[pl.ds(j, LANES)] = dst_b[pl.ds(el + j, LANES)]

    def compute(esed):
        @pl.loop(0, EB)
        def _(r):
            for g in range(HW // LANES):
                sl = pl.ds(g * LANES, LANES)
                mv = eg_b[r, sl] + esed[r, sl] + esed[EB + r, sl]
                m_b[r, sl] = mv
                sg = 1.0 / (1.0 + jnp.exp(-mv))
                half = csgA if g < CW // LANES else csgB
                co = (g % (CW // LANES)) * LANES
                half[r, pl.ds(co, LANES)] = sg * bs_b[r, sl]
                half[r, pl.ds(CW + co, LANES)] = sg

    def fire_eg(eoff):
        pltpu.async_copy(egt.at[cid, pl.ds(ebase + eoff, EB)], eg_b, sem3)

    def fire_m(eoff):
        return pltpu.async_copy(
            m_b, m_hbm.at[pl.ds(ebase + eoff, EB), pl.ds(cid * HW, HW)], semw)

    def drain(descr_src, descr_dst, sem):
        pltpu.make_async_copy(descr_src, descr_dst, sem).wait()

    zero_acc()

    @pl.loop(0, NPAIR)
    def _(pr):
        lt = 2 * (pr % NPPS)
        sb = (pr // NPPS) * STG
        ea = sb + lt * EB
        eb2 = ea + EB

        @pl.when(pr % NPPS == 0)
        def _():
            pltpu.sync_copy(src.at[pl.ds(ebase + sb, STG)], src_b)
            pltpu.sync_copy(dst.at[pl.ds(ebase + sb, STG)], dst_b)
            build_idx(eidxA, 0)
            pltpu.async_copy(tab.at[eidxA], esedA, sem0)
            build_bsi(0)
            pltpu.async_copy(tab.at[bsi], bs_b, sem2)
            fire_eg(ea)

        build_idx(eidxB, lt + 1)
        pltpu.async_copy(tab.at[eidxB], esedB, sem1)
        drain(tab.at[eidxA], esedA, sem0)
        drain(tab.at[bsi], bs_b, sem2)
        drain(egt.at[cid, pl.ds(ebase + ea, EB)], eg_b, sem3)
        compute(esedA)
        wm = fire_m(ea)
        ws = pltpu.async_copy(csgB, spill_hbm.at[cid, pl.ds(ebase + ea, EB)],
                              semw)
        build_bsi(lt + 1)
        pltpu.async_copy(tab.at[bsi], bs_b, sem2)
        fire_eg(eb2)
        build_dsc(lt)
        pltpu.sync_copy(csgA, acc.at[dsc], add=True)
        wm.wait()
        ws.wait()

        @pl.when(pr % NPPS < NPPS - 1)
        def _():
            build_idx(eidxA, lt + 2)
            pltpu.async_copy(tab.at[eidxA], esedA, sem0)

        drain(tab.at[eidxB], esedB, sem1)
        drain(tab.at[bsi], bs_b, sem2)
        drain(egt.at[cid, pl.ds(ebase + eb2, EB)], eg_b, sem3)
        compute(esedB)
        wm2 = fire_m(eb2)
        ws2 = pltpu.async_copy(csgB, spill_hbm.at[cid, pl.ds(ebase + eb2, EB)],
                               semw)

        @pl.when(pr % NPPS < NPPS - 1)
        def _():
            build_bsi(lt + 2)
            pltpu.async_copy(tab.at[bsi], bs_b, sem2)
            fire_eg(ea + 2 * EB)

        build_dsc(lt + 1)
        pltpu.sync_copy(csgA, acc.at[dsc], add=True)
        wm2.wait()
        ws2.wait()

    writeout_acc(s2a_hbm)

    B2 = 2 * EB
    NB2 = EPW // B2
    BPS = STG // B2

    def build_dsc80(buf, t):
        el = (t % BPS) * B2
        for j in range(0, B2, LANES):
            buf[pl.ds(j, LANES)] = dst_b[pl.ds(el + j, LANES)]

    def reload_if_stage_top(t):
        @pl.when(t % BPS == 0)
        def _():
            pltpu.sync_copy(
                dst.at[pl.ds(ebase + (t // BPS) * STG, STG)], dst_b)

    zero_acc()

    pltpu.async_copy(spill_hbm.at[cid, pl.ds(ebase, B2)], esedA, sem0)

    @pl.loop(0, NB2 // 2)
    def _(pr):
        t0 = 2 * pr
        e0 = t0 * B2

        reload_if_stage_top(t0)
        drain(spill_hbm.at[cid, pl.ds(ebase + e0, B2)], esedA, sem0)
        pltpu.async_copy(spill_hbm.at[cid, pl.ds(ebase + e0 + B2, B2)],
                         esedB, sem1)
        build_dsc80(eidxA, t0)
        pltpu.sync_copy(esedA, acc.at[eidxA], add=True)

        reload_if_stage_top(t0 + 1)
        drain(spill_hbm.at[cid, pl.ds(ebase + e0 + B2, B2)], esedB, sem1)
        pltpu.async_copy(spill_hbm.at[cid, pl.ds(ebase + e0 + 2 * B2, B2)],
                         esedA, sem0)
        build_dsc80(eidxB, t0 + 1)
        pltpu.sync_copy(esedB, acc.at[eidxB], add=True)

    drain(spill_hbm.at[cid, pl.ds(ebase + (NB2 - 1) * B2, B2)], esedA, sem0)
    build_dsc80(eidxA, NB2 - 1)
    pltpu.sync_copy(esedA, acc.at[eidxA], add=True)

    writeout_acc(s2b_hbm)


def _node_proj(node_feats, W_sg, b_sg, W_dg, b_dg, W_du, b_du, W_su, b_su):
    wspec = pl.BlockSpec((D, D), lambda i: (0, 0))
    bspec = pl.BlockSpec((1, D), lambda i: (0, 0))
    return pl.pallas_call(
        _node_proj_body,
        grid=(N // _NODE_BLK,),
        in_specs=[
            pl.BlockSpec((_NODE_BLK, D), lambda i: (i, 0)),
            wspec, bspec, wspec, bspec, wspec, bspec, wspec, bspec,
        ],
        out_specs=[
            pl.BlockSpec((3 * NC, _NODE_BLK, HW), lambda i: (0, i, 0)),
            pl.BlockSpec((_NODE_BLK, D), lambda i: (i, 0)),
        ],
        out_shape=[
            jax.ShapeDtypeStruct((3 * NC, N, HW), jnp.float32),
            jax.ShapeDtypeStruct((N, D), jnp.float32),
        ],
    )(node_feats, W_sg, b_sg, W_dg, b_dg, W_du, b_du, W_su, b_su)


def _edge_proj(edge_feats, W_eg, b_eg):
    return pl.pallas_call(
        _edge_proj_body,
        grid=(E // _EDGE_BLK,),
        in_specs=[
            pl.BlockSpec((_EDGE_BLK, D), lambda i: (i, 0)),
            pl.BlockSpec((D, D), lambda i: (0, 0)),
            pl.BlockSpec((1, D), lambda i: (0, 0)),
        ],
        out_specs=pl.BlockSpec((NC, _EDGE_BLK, HW), lambda i: (0, i, 0)),
        out_shape=jax.ShapeDtypeStruct((NC, E, HW), jnp.float32),
    )(edge_feats, W_eg, b_eg)


def _combine(xsu, s2a, s2b):
    return pl.pallas_call(
        _combine_body,
        grid=(N // _NODE_BLK,),
        in_specs=[
            pl.BlockSpec((_NODE_BLK, D), lambda i: (i, 0)),
            pl.BlockSpec((NC, _NODE_BLK, HW), lambda i: (0, i, 0)),
            pl.BlockSpec((NC, _NODE_BLK, HW), lambda i: (0, i, 0)),
        ],
        out_specs=pl.BlockSpec((_NODE_BLK, D), lambda i: (i, 0)),
        out_shape=jax.ShapeDtypeStruct((N, D), jnp.float32),
    )(xsu, s2a, s2b)


def _sc_edge(tab, egt, src, dst):
    mesh = plsc.VectorSubcoreMesh(core_axis_name="c", subcore_axis_name="s")
    f = pl.kernel(
        _sc_edge_body,
        mesh=mesh,
        out_type=[
            jax.ShapeDtypeStruct((E, D), jnp.float32),
            jax.ShapeDtypeStruct((NC, N, HW), jnp.float32),
            jax.ShapeDtypeStruct((NC, N, HW), jnp.float32),
            jax.ShapeDtypeStruct((NC, E, HW), jnp.float32),
        ],
        scratch_types=[
            pltpu.VMEM((STG,), jnp.int32),
            pltpu.VMEM((STG,), jnp.int32),
            pltpu.VMEM((2 * EB,), jnp.int32),
            pltpu.VMEM((2 * EB,), jnp.int32),
            pltpu.VMEM((EB,), jnp.int32),
            pltpu.VMEM((EB,), jnp.int32),
            pltpu.VMEM((2 * EB, HW), jnp.float32),
            pltpu.VMEM((2 * EB, HW), jnp.float32),
            pltpu.VMEM((EB, HW), jnp.float32),
            pltpu.VMEM((EB, HW), jnp.float32),
            pltpu.VMEM((EB, HW), jnp.float32),
            pltpu.VMEM((EB, HW), jnp.float32),
            pltpu.VMEM((EB, HW), jnp.float32),
            pltpu.VMEM_SHARED((N, HW), jnp.float32),
            pltpu.SemaphoreType.DMA,
            pltpu.SemaphoreType.DMA,
            pltpu.SemaphoreType.DMA,
            pltpu.SemaphoreType.DMA,
            pltpu.SemaphoreType.DMA,
        ],
    )
    return f(tab, egt, src, dst)


@jax.jit
def kernel(node_feats, edge_feats, edge_index, W_sg, b_sg, W_dg, b_dg,
           W_eg, b_eg, W_su, b_su, W_du, b_du):
    src = edge_index[0].astype(jnp.int32)
    dst = edge_index[1].astype(jnp.int32)

    tab, xsu = _node_proj(node_feats, W_sg, b_sg.reshape(1, D),
                          W_dg, b_dg.reshape(1, D), W_du, b_du.reshape(1, D),
                          W_su, b_su.reshape(1, D))
    egt = _edge_proj(edge_feats, W_eg, b_eg.reshape(1, D))

    m, s2a, s2b, _spill = _sc_edge(
        tab.reshape(3 * NC * N, HW), egt, src, dst)
    x = _combine(xsu, s2a, s2b)
    return (x, m)


# speedup vs baseline: 1.1392x; 1.0035x over previous
import jax
import jax.numpy as jnp
from jax import lax
from jax.experimental import pallas as pl
from jax.experimental.pallas import tpu as pltpu
from jax.experimental.pallas import tpu_sc as plsc

N = 10000
E = 160000
D = 256
NC = 2
NS = 16
LANES = 16
HW = 128
CW = 64
EPW = E // NS
EB = 40
STG = 2000
NPPS = STG // (2 * EB)
NPAIR = EPW // (2 * EB)
NPW = N // NS
_TAIL = (0, 16, 24)

_NODE_BLK = 2000
_EDGE_BLK = 8000


def _node_proj_body(x_ref, wsg_ref, bsg_ref, wdg_ref, bdg_ref, wdu_ref,
                    bdu_ref, wsu_ref, bsu_ref, t_ref, xsu_ref):
    x = x_ref[...]
    for q, (w, b) in enumerate(((wsg_ref, bsg_ref), (wdg_ref, bdg_ref),
                                (wdu_ref, bdu_ref))):
        g = jnp.dot(x, w[...], preferred_element_type=jnp.float32) + b[...]
        for h in range(NC):
            t_ref[q * NC + h] = g[:, h * HW:(h + 1) * HW]
    xsu_ref[...] = (
        jnp.dot(x, wsu_ref[...], preferred_element_type=jnp.float32) + bsu_ref[...]
    )


def _edge_proj_body(x_ref, w_ref, b_ref, out_ref):
    g = jnp.dot(x_ref[...], w_ref[...], preferred_element_type=jnp.float32) + b_ref[...]
    for h in range(NC):
        out_ref[h] = g[:, h * HW:(h + 1) * HW]


def _combine_body(xsu_ref, s2a_ref, s2b_ref, x_ref):
    for c in range(D // CW):
        h, ph = c // 2, c % 2
        sref = s2a_ref if ph == 0 else s2b_ref
        x_ref[:, c * CW:(c + 1) * CW] = (
            xsu_ref[:, c * CW:(c + 1) * CW]
            + sref[h][:, 0:CW] / (sref[h][:, CW:HW] + 1e-6)
        )


def _sc_edge_body(tab, egt, src, dst, m_hbm, s2a_hbm, s2b_hbm, spill_hbm,
                  src_b, dst_b, eidxA, eidxB, bsi, dsc,
                  esedA, esedB, bs_b, eg_b, m_b, csgA, csgB,
                  acc, sem0, sem1, sem2, sem3, semw):
    cid = lax.axis_index("c")
    sid = lax.axis_index("s")
    ebase = sid * EPW

    es_off = (0 * NC + cid) * N
    ed_off = (1 * NC + cid) * N
    bs_off = (2 * NC + cid) * N

    def zero_acc():
        @pl.loop(0, EB)
        def _(r):
            for g in range(HW // LANES):
                csgA[r, pl.ds(g * LANES, LANES)] = jnp.zeros((LANES,), jnp.float32)

        @pl.loop(0, 15)
        def _(z):
            pltpu.sync_copy(csgA, acc.at[pl.ds(sid * NPW + z * EB, EB)])

        pltpu.sync_copy(csgA.at[pl.ds(0, 25)],
                        acc.at[pl.ds(sid * NPW + 600, 25)])
        plsc.subcore_barrier()

    def writeout_acc(dst_hbm):
        plsc.subcore_barrier()
        @pl.when(sid < NS - 1)
        def _():
            pltpu.sync_copy(acc.at[pl.ds(sid * 640, 640)],
                            dst_hbm.at[cid, pl.ds(sid * 640, 640)])

        @pl.when(sid == NS - 1)
        def _():
            pltpu.sync_copy(acc.at[pl.ds(9600, 400)],
                            dst_hbm.at[cid, pl.ds(9600, 400)])

        plsc.subcore_barrier()

    def build_idx(ebuf, lt):
        el = lt * EB
        for j in _TAIL:
            ebuf[pl.ds(j, LANES)] = src_b[pl.ds(el + j, LANES)] + es_off
            ebuf[pl.ds(EB + j, LANES)] = dst_b[pl.ds(el + j, LANES)] + ed_off

    def build_bsi(lt):
        el = lt * EB
        for j in _TAIL:
            bsi[pl.ds(j, LANES)] = src_b[pl.ds(el + j, LANES)] + bs_off

    def build_dsc(lt):
        el = lt * EB
        for j in _TAIL:
            dsc[pl.ds(j, LANES)] = dst_b[pl.ds(el + j, LANES)]

    def compute(esed):
        @pl.loop(0, EB)
        def _(r):
            for g in range(HW // LANES):
                sl = pl.ds(g * LANES, LANES)
                mv = eg_b[r, sl] + esed[r, sl] + esed[EB + r, sl]
                m_b[r, sl] = mv
                sg = 1.0 / (1.0 + jnp.exp(-mv))
                half = csgA if g < CW // LANES else csgB
                co = (g % (CW // LANES)) * LANES
                half[r, pl.ds(co, LANES)] = sg * bs_b[r, sl]
                half[r, pl.ds(CW + co, LANES)] = sg

    def fire_eg(eoff):
        pltpu.async_copy(egt.at[cid, pl.ds(ebase + eoff, EB)], eg_b, sem3)

    def fire_m(eoff):
        return pltpu.async_copy(
            m_b, m_hbm.at[pl.ds(ebase + eoff, EB), pl.ds(cid * HW, HW)], semw)

    def drain(descr_src, descr_dst, sem):
        pltpu.make_async_copy(descr_src, descr_dst, sem).wait()

    zero_acc()

    @pl.loop(0, NPAIR)
    def _(pr):
        lt = 2 * (pr % NPPS)
        sb = (pr // NPPS) * STG
        ea = sb + lt * EB
        eb2 = ea + EB

        @pl.when(pr % NPPS == 0)
        def _():
            pltpu.sync_copy(src.at[pl.ds(ebase + sb, STG)], src_b)
            pltpu.sync_copy(dst.at[pl.ds(ebase + sb, STG)], dst_b)
            build_idx(eidxA, 0)
            pltpu.async_copy(tab.at[eidxA], esedA, sem0)
            build_bsi(0)
            pltpu.async_copy(tab.at[bsi], bs_b, sem2)
            fire_eg(ea)

        build_idx(eidxB, lt + 1)
        pltpu.async_copy(tab.at[eidxB], esedB, sem1)
        drain(tab.at[eidxA], esedA, sem0)
        drain(tab.at[bsi], bs_b, sem2)
        drain(egt.at[cid, pl.ds(ebase + ea, EB)], eg_b, sem3)
        compute(esedA)
        wm = fire_m(ea)
        ws = pltpu.async_copy(csgB, spill_hbm.at[cid, pl.ds(ebase + ea, EB)],
                              semw)
        build_bsi(lt + 1)
        pltpu.async_copy(tab.at[bsi], bs_b, sem2)
        fire_eg(eb2)
        build_dsc(lt)
        pltpu.sync_copy(csgA, acc.at[dsc], add=True)
        wm.wait()
        ws.wait()

        @pl.when(pr % NPPS < NPPS - 1)
        def _():
            build_idx(eidxA, lt + 2)
            pltpu.async_copy(tab.at[eidxA], esedA, sem0)

        drain(tab.at[eidxB], esedB, sem1)
        drain(tab.at[bsi], bs_b, sem2)
        drain(egt.at[cid, pl.ds(ebase + eb2, EB)], eg_b, sem3)
        compute(esedB)
        wm2 = fire_m(eb2)
        ws2 = pltpu.async_copy(csgB, spill_hbm.at[cid, pl.ds(ebase + eb2, EB)],
                               semw)

        @pl.when(pr % NPPS < NPPS - 1)
        def _():
            build_bsi(lt + 2)
            pltpu.async_copy(tab.at[bsi], bs_b, sem2)
            fire_eg(ea + 2 * EB)

        build_dsc(lt + 1)
        pltpu.sync_copy(csgA, acc.at[dsc], add=True)
        wm2.wait()
        ws2.wait()

    writeout_acc(s2a_hbm)

    B2 = 2 * EB
    NB2 = EPW // B2
    BPS = STG // B2

    def build_dsc80(buf, t):
        el = (t % BPS) * B2
        for j in range(0, B2, LANES):
            buf[pl.ds(j, LANES)] = dst_b[pl.ds(el + j, LANES)]

    def reload_if_stage_top(t):
        @pl.when(t % BPS == 0)
        def _():
            pltpu.sync_copy(
                dst.at[pl.ds(ebase + (t // BPS) * STG, STG)], dst_b)

    zero_acc()

    pltpu.async_copy(spill_hbm.at[cid, pl.ds(ebase, B2)], esedA, sem0)

    @pl.loop(0, NB2 // 2)
    def _(pr):
        t0 = 2 * pr
        e0 = t0 * B2

        reload_if_stage_top(t0)
        drain(spill_hbm.at[cid, pl.ds(ebase + e0, B2)], esedA, sem0)
        pltpu.async_copy(spill_hbm.at[cid, pl.ds(ebase + e0 + B2, B2)],
                         esedB, sem1)
        build_dsc80(eidxA, t0)
        pltpu.sync_copy(esedA, acc.at[eidxA], add=True)

        reload_if_stage_top(t0 + 1)
        drain(spill_hbm.at[cid, pl.ds(ebase + e0 + B2, B2)], esedB, sem1)
        pltpu.async_copy(spill_hbm.at[cid, pl.ds(ebase + e0 + 2 * B2, B2)],
                         esedA, sem0)
        build_dsc80(eidxB, t0 + 1)
        pltpu.sync_copy(esedB, acc.at[eidxB], add=True)

    drain(spill_hbm.at[cid, pl.ds(ebase + (NB2 - 1) * B2, B2)], esedA, sem0)
    build_dsc80(eidxA, NB2 - 1)
    pltpu.sync_copy(esedA, acc.at[eidxA], add=True)

    writeout_acc(s2b_hbm)


def _node_proj(node_feats, W_sg, b_sg, W_dg, b_dg, W_du, b_du, W_su, b_su):
    wspec = pl.BlockSpec((D, D), lambda i: (0, 0))
    bspec = pl.BlockSpec((1, D), lambda i: (0, 0))
    return pl.pallas_call(
        _node_proj_body,
        grid=(N // _NODE_BLK,),
        in_specs=[
            pl.BlockSpec((_NODE_BLK, D), lambda i: (i, 0)),
            wspec, bspec, wspec, bspec, wspec, bspec, wspec, bspec,
        ],
        out_specs=[
            pl.BlockSpec((3 * NC, _NODE_BLK, HW), lambda i: (0, i, 0)),
            pl.BlockSpec((_NODE_BLK, D), lambda i: (i, 0)),
        ],
        out_shape=[
            jax.ShapeDtypeStruct((3 * NC, N, HW), jnp.float32),
            jax.ShapeDtypeStruct((N, D), jnp.float32),
        ],
    )(node_feats, W_sg, b_sg, W_dg, b_dg, W_du, b_du, W_su, b_su)


def _edge_proj(edge_feats, W_eg, b_eg):
    return pl.pallas_call(
        _edge_proj_body,
        grid=(E // _EDGE_BLK,),
        in_specs=[
            pl.BlockSpec((_EDGE_BLK, D), lambda i: (i, 0)),
            pl.BlockSpec((D, D), lambda i: (0, 0)),
            pl.BlockSpec((1, D), lambda i: (0, 0)),
        ],
        out_specs=pl.BlockSpec((NC, _EDGE_BLK, HW), lambda i: (0, i, 0)),
        out_shape=jax.ShapeDtypeStruct((NC, E, HW), jnp.float32),
    )(edge_feats, W_eg, b_eg)


def _combine(xsu, s2a, s2b):
    return pl.pallas_call(
        _combine_body,
        grid=(N // _NODE_BLK,),
        in_specs=[
            pl.BlockSpec((_NODE_BLK, D), lambda i: (i, 0)),
            pl.BlockSpec((NC, _NODE_BLK, HW), lambda i: (0, i, 0)),
            pl.BlockSpec((NC, _NODE_BLK, HW), lambda i: (0, i, 0)),
        ],
        out_specs=pl.BlockSpec((_NODE_BLK, D), lambda i: (i, 0)),
        out_shape=jax.ShapeDtypeStruct((N, D), jnp.float32),
    )(xsu, s2a, s2b)


def _sc_edge(tab, egt, src, dst):
    mesh = plsc.VectorSubcoreMesh(core_axis_name="c", subcore_axis_name="s")
    f = pl.kernel(
        _sc_edge_body,
        mesh=mesh,
        out_type=[
            jax.ShapeDtypeStruct((E, D), jnp.float32),
            jax.ShapeDtypeStruct((NC, N, HW), jnp.float32),
            jax.ShapeDtypeStruct((NC, N, HW), jnp.float32),
            jax.ShapeDtypeStruct((NC, E, HW), jnp.float32),
        ],
        scratch_types=[
            pltpu.VMEM((STG,), jnp.int32),
            pltpu.VMEM((STG,), jnp.int32),
            pltpu.VMEM((2 * EB,), jnp.int32),
            pltpu.VMEM((2 * EB,), jnp.int32),
            pltpu.VMEM((EB,), jnp.int32),
            pltpu.VMEM((EB,), jnp.int32),
            pltpu.VMEM((2 * EB, HW), jnp.float32),
            pltpu.VMEM((2 * EB, HW), jnp.float32),
            pltpu.VMEM((EB, HW), jnp.float32),
            pltpu.VMEM((EB, HW), jnp.float32),
            pltpu.VMEM((EB, HW), jnp.float32),
            pltpu.VMEM((EB, HW), jnp.float32),
            pltpu.VMEM((EB, HW), jnp.float32),
            pltpu.VMEM_SHARED((N, HW), jnp.float32),
            pltpu.SemaphoreType.DMA,
            pltpu.SemaphoreType.DMA,
            pltpu.SemaphoreType.DMA,
            pltpu.SemaphoreType.DMA,
            pltpu.SemaphoreType.DMA,
        ],
    )
    return f(tab, egt, src, dst)


@jax.jit
def kernel(node_feats, edge_feats, edge_index, W_sg, b_sg, W_dg, b_dg,
           W_eg, b_eg, W_su, b_su, W_du, b_du):
    src = edge_index[0].astype(jnp.int32)
    dst = edge_index[1].astype(jnp.int32)

    tab, xsu = _node_proj(node_feats, W_sg, b_sg.reshape(1, D),
                          W_dg, b_dg.reshape(1, D), W_du, b_du.reshape(1, D),
                          W_su, b_su.reshape(1, D))
    egt = _edge_proj(edge_feats, W_eg, b_eg.reshape(1, D))

    m, s2a, s2b, _spill = _sc_edge(
        tab.reshape(3 * NC * N, HW), egt, src, dst)
    x = _combine(xsu, s2a, s2b)
    return (x, m)
